# Initial kernel scaffold; baseline (speedup 1.0000x reference)
import jax
import jax.numpy as jnp
from jax.experimental import pallas as pl


def kernel(preds, targets):
    raise NotImplementedError("write your pallas kernel here")


# speedup vs baseline: 5.4629x; 5.4629x over previous
import functools
import math

import jax
import jax.numpy as jnp
from jax import lax
from jax.experimental import pallas as pl
from jax.experimental.pallas import tpu as pltpu

_EPS = 0.05
_BLK = 256


def _auc_kernel(preds_ref, predsT_ref, trow_ref, tcol_ref, out_ref,
                oh_scr, e4_scr, e4m_scr, a_scr, b_scr, fac_scr):
    B, C = preds_ref.shape
    i = pl.program_id(0)
    c2 = math.exp(_EPS)
    c1 = 1.0 + math.exp(2 * _EPS)

    @pl.when(i == 0)
    def _init():
        z = preds_ref[...]
        e = jnp.exp(z - jnp.max(z, axis=1, keepdims=True))
        p = e / jnp.sum(e, axis=1, keepdims=True)
        cls = lax.broadcasted_iota(jnp.int32, (1, C), 1)
        oh = (tcol_ref[...] == cls).astype(jnp.float32)
        oh_scr[...] = oh
        counts = jnp.sum(oh, axis=0, keepdims=True)
        denom = counts * (float(B) - counts)
        fac = jnp.where(denom > 0.0, 1.0 / denom, 0.0)
        fac_scr[...] = jnp.sum(oh * fac, axis=1, keepdims=True)
        q = jnp.sum(oh * p, axis=1, keepdims=True)
        a_scr[...] = c2 * jnp.exp(4.0 - 4.0 * q)
        b_scr[...] = c2 * jnp.exp(4.0 * q - 4.0)
        zt = predsT_ref[...]
        et = jnp.exp(zt - jnp.max(zt, axis=0, keepdims=True))
        pt = et / jnp.sum(et, axis=0, keepdims=True)
        e4_scr[...] = jnp.exp(4.0 * pt)
        e4m_scr[...] = jnp.exp(-4.0 * pt)
        out_ref[...] = jnp.zeros_like(out_ref)

    ds = pl.ds(i * _BLK, _BLK)
    oh_blk = oh_scr[ds, :]
    eg = lax.dot(oh_blk, e4_scr[...], precision=lax.Precision.HIGHEST)
    emg = lax.dot(oh_blk, e4m_scr[...], precision=lax.Precision.HIGHEST)
    a = a_scr[ds, :]
    b = b_scr[ds, :]
    fc = fac_scr[ds, :]
    ta = tcol_ref[ds, :]
    f = jnp.log(c1 + a * eg + b * emg)
    masked = jnp.where(ta != trow_ref[...], f, 0.0)
    rowsum = jnp.sum(masked, axis=1, keepdims=True)
    out_ref[...] += jnp.sum(rowsum * fc, keepdims=True).reshape(1, 1)


@functools.partial(jax.jit, static_argnames=("interpret",))
def kernel(preds, targets, interpret=False):
    B, C = preds.shape
    t32 = targets.astype(jnp.int32)
    preds_t = preds.T
    t_row = t32.reshape(1, B)
    t_col = t32.reshape(B, 1)
    out = pl.pallas_call(
        _auc_kernel,
        grid=(B // _BLK,),
        in_specs=[
            pl.BlockSpec((B, C), lambda i: (0, 0)),
            pl.BlockSpec((C, B), lambda i: (0, 0)),
            pl.BlockSpec((1, B), lambda i: (0, 0)),
            pl.BlockSpec((B, 1), lambda i: (0, 0)),
        ],
        out_specs=pl.BlockSpec((1, 1), lambda i: (0, 0)),
        out_shape=jax.ShapeDtypeStruct((1, 1), jnp.float32),
        scratch_shapes=[
            pltpu.VMEM((B, C), jnp.float32),
            pltpu.VMEM((C, B), jnp.float32),
            pltpu.VMEM((C, B), jnp.float32),
            pltpu.VMEM((B, 1), jnp.float32),
            pltpu.VMEM((B, 1), jnp.float32),
            pltpu.VMEM((B, 1), jnp.float32),
        ],
        interpret=interpret,
    )(preds, preds_t, t_row, t_col)
    return out.reshape((1,))


# speedup vs baseline: 8.4115x; 1.5397x over previous
import functools
import math

import jax
import jax.numpy as jnp
from jax import lax
from jax.experimental import pallas as pl
from jax.experimental.pallas import tpu as pltpu

_EPS = 0.05
_BLK = 256


def _auc_kernel(preds_ref, predsT_ref, trow_ref, tcol_ref, out_ref,
                lhs_scr, rhs_scr, fac_scr):
    B, C = preds_ref.shape
    i = pl.program_id(0)
    c2 = math.exp(_EPS)
    c1 = 1.0 + math.exp(2 * _EPS)

    @pl.when(i == 0)
    def _init():
        z = preds_ref[...]
        e = jnp.exp(z - jnp.max(z, axis=1, keepdims=True))
        p = e / jnp.sum(e, axis=1, keepdims=True)
        cls = lax.broadcasted_iota(jnp.int32, (1, C), 1)
        oh = (tcol_ref[...] == cls).astype(jnp.float32)
        counts = jnp.sum(oh, axis=0, keepdims=True)
        denom = counts * (float(B) - counts)
        fac = jnp.where(denom > 0.0, 1.0 / denom, 0.0)
        fac_scr[...] = jnp.sum(oh * fac, axis=1, keepdims=True)
        q = jnp.sum(oh * p, axis=1, keepdims=True)
        a = c2 * jnp.exp(4.0 - 4.0 * q)
        b = c2 * jnp.exp(4.0 * q - 4.0)
        lhs_scr[...] = jnp.concatenate(
            [oh * a, oh * b, jnp.ones((B, 1), jnp.float32),
             (1.0 - c1) * oh, jnp.zeros((B, 1), jnp.float32)], axis=1)
        zt = predsT_ref[...]
        et = jnp.exp(zt - jnp.max(zt, axis=0, keepdims=True))
        pt = et / jnp.sum(et, axis=0, keepdims=True)
        clsr = lax.broadcasted_iota(jnp.int32, (C, B), 0)
        same = trow_ref[...] == clsr
        e4 = jnp.where(same, 0.0, jnp.exp(4.0 * pt))
        e4m = jnp.where(same, 0.0, jnp.exp(-4.0 * pt))
        rhs_scr[...] = jnp.concatenate(
            [e4, e4m, jnp.full((1, B), c1, jnp.float32),
             same.astype(jnp.float32), jnp.zeros((1, B), jnp.float32)], axis=0)
        out_ref[...] = jnp.zeros_like(out_ref)

    ds = pl.ds(i * _BLK, _BLK)
    arg = lax.dot(lhs_scr[ds, :], rhs_scr[...],
                  precision=lax.Precision.HIGHEST)
    rowsum = jnp.sum(jnp.log(arg), axis=1, keepdims=True)
    out_ref[...] += jnp.sum(rowsum * fac_scr[ds, :], keepdims=True).reshape(1, 1)


@functools.partial(jax.jit, static_argnames=("interpret",))
def kernel(preds, targets, interpret=False):
    B, C = preds.shape
    K = 2 * C + 2 + C
    t32 = targets.astype(jnp.int32)
    preds_t = preds.T
    t_row = t32.reshape(1, B)
    t_col = t32.reshape(B, 1)
    out = pl.pallas_call(
        _auc_kernel,
        grid=(B // _BLK,),
        in_specs=[
            pl.BlockSpec((B, C), lambda i: (0, 0)),
            pl.BlockSpec((C, B), lambda i: (0, 0)),
            pl.BlockSpec((1, B), lambda i: (0, 0)),
            pl.BlockSpec((B, 1), lambda i: (0, 0)),
        ],
        out_specs=pl.BlockSpec((1, 1), lambda i: (0, 0)),
        out_shape=jax.ShapeDtypeStruct((1, 1), jnp.float32),
        scratch_shapes=[
            pltpu.VMEM((B, K), jnp.float32),
            pltpu.VMEM((K, B), jnp.float32),
            pltpu.VMEM((B, 1), jnp.float32),
        ],
        interpret=interpret,
    )(preds, preds_t, t_row, t_col)
    return out.reshape((1,))


# speedup vs baseline: 11.4106x; 1.3565x over previous
import functools
import math

import jax
import jax.numpy as jnp
from jax import lax
from jax.experimental import pallas as pl
from jax.experimental.pallas import tpu as pltpu

_EPS = 0.05
_BLK = 256


def _auc_kernel(predsT_ref, trow_ref, out_ref, m1_scr, m2_scr, fac_scr):
    C, B = predsT_ref.shape
    i = pl.program_id(0)
    c2 = math.exp(_EPS)
    c1 = 1.0 + math.exp(2 * _EPS)

    @pl.when(i == 0)
    def _init():
        zt = predsT_ref[...]
        et = jnp.exp(zt - jnp.max(zt, axis=0, keepdims=True))
        pt = et / jnp.sum(et, axis=0, keepdims=True)
        clsr = lax.broadcasted_iota(jnp.int32, (C, B), 0)
        same = trow_ref[...] == clsr
        ohT = same.astype(jnp.float32)
        counts = jnp.sum(ohT, axis=1, keepdims=True)
        denom = counts * (float(B) - counts)
        facc = jnp.where(denom > 0.0, 1.0 / denom, 0.0)
        fac_scr[...] = jnp.sum(ohT * facc, axis=0, keepdims=True)
        q = jnp.sum(ohT * pt, axis=0, keepdims=True)
        a = c2 * jnp.exp(4.0 - 4.0 * q)
        b = c2 * jnp.exp(4.0 * q - 4.0)
        e4 = jnp.where(same, 0.0, jnp.exp(4.0 * pt))
        e4m = jnp.where(same, 0.0, jnp.exp(-4.0 * pt))
        ones = jnp.ones((1, B), jnp.float32)
        zeros = jnp.zeros((1, B), jnp.float32)
        m1_scr[...] = jnp.concatenate(
            [e4, e4m, c1 * ones, ohT, zeros], axis=0)
        m2_scr[...] = jnp.concatenate(
            [a * ohT, b * ohT, ones, (1.0 - c1) * ohT, zeros], axis=0)
        out_ref[...] = jnp.zeros_like(out_ref)

    ds = pl.ds(i * _BLK, _BLK)
    arg = lax.dot_general(
        m1_scr[...], m2_scr[:, ds],
        dimension_numbers=(((0,), (0,)), ((), ())),
        precision=lax.Precision.HIGHEST)
    colsum = jnp.sum(jnp.log(arg), axis=0, keepdims=True)
    out_ref[...] += jnp.sum(colsum * fac_scr[:, ds],
                            keepdims=True).reshape(1, 1)


@functools.partial(jax.jit, static_argnames=("interpret",))
def kernel(preds, targets, interpret=False):
    B, C = preds.shape
    K = 3 * C + 2
    preds_t = preds.T
    t_row = targets.astype(jnp.int32).reshape(1, B)
    out = pl.pallas_call(
        _auc_kernel,
        grid=(B // _BLK,),
        in_specs=[
            pl.BlockSpec((C, B), lambda i: (0, 0)),
            pl.BlockSpec((1, B), lambda i: (0, 0)),
        ],
        out_specs=pl.BlockSpec((1, 1), lambda i: (0, 0)),
        out_shape=jax.ShapeDtypeStruct((1, 1), jnp.float32),
        scratch_shapes=[
            pltpu.VMEM((K, B), jnp.float32),
            pltpu.VMEM((K, B), jnp.float32),
            pltpu.VMEM((1, B), jnp.float32),
        ],
        interpret=interpret,
    )(preds_t, t_row)
    return out.reshape((1,))


# speedup vs baseline: 25.6782x; 2.2504x over previous
import functools
import math

import jax
import jax.numpy as jnp
from jax import lax
from jax.experimental import pallas as pl
from jax.experimental.pallas import tpu as pltpu

_EPS = 0.05
_BLK = 256


def _auc_kernel(predsT_ref, trow_ref, out_ref, m1_scr, m2_scr, fac_scr):
    C, B = predsT_ref.shape
    i = pl.program_id(0)
    c2 = math.exp(_EPS)
    c1 = 1.0 + math.exp(2 * _EPS)

    @pl.when(i == 0)
    def _init():
        zt = predsT_ref[...]
        et = jnp.exp(zt - jnp.max(zt, axis=0, keepdims=True))
        pt = et / jnp.sum(et, axis=0, keepdims=True)
        clsr = lax.broadcasted_iota(jnp.int32, (C, B), 0)
        same = trow_ref[...] == clsr
        ohT = same.astype(jnp.float32)
        counts = jnp.sum(ohT, axis=1, keepdims=True)
        denom = counts * (float(B) - counts)
        facc = jnp.where(denom > 0.0, 1.0 / denom, 0.0)
        fac_scr[...] = jnp.sum(ohT * facc, axis=0, keepdims=True)
        q = jnp.sum(ohT * pt, axis=0, keepdims=True)
        a = c2 * jnp.exp(4.0 - 4.0 * q)
        b = c2 * jnp.exp(4.0 * q - 4.0)
        e4 = jnp.where(same, 0.0, jnp.exp(4.0 * pt))
        e4m = jnp.where(same, 0.0, jnp.exp(-4.0 * pt))
        ones = jnp.ones((1, B), jnp.float32)
        zeros = jnp.zeros((1, B), jnp.float32)
        m1_scr[...] = jnp.concatenate(
            [e4, e4m, c1 * ones, ohT, zeros], axis=0)
        m2_scr[...] = jnp.concatenate(
            [a * ohT, b * ohT, ones, (1.0 - c1) * ohT, zeros], axis=0)
        out_ref[...] = jnp.zeros_like(out_ref)

    ds = pl.ds(i * _BLK, _BLK)
    arg = lax.dot_general(
        m1_scr[...], m2_scr[:, ds],
        dimension_numbers=(((0,), (0,)), ((), ())),
        precision=lax.Precision.DEFAULT)
    colsum = jnp.sum(jnp.log(arg), axis=0, keepdims=True)
    out_ref[...] += jnp.sum(colsum * fac_scr[:, ds],
                            keepdims=True).reshape(1, 1)


@functools.partial(jax.jit, static_argnames=("interpret",))
def kernel(preds, targets, interpret=False):
    B, C = preds.shape
    K = 3 * C + 2
    preds_t = preds.T
    t_row = targets.astype(jnp.int32).reshape(1, B)
    out = pl.pallas_call(
        _auc_kernel,
        grid=(B // _BLK,),
        in_specs=[
            pl.BlockSpec((C, B), lambda i: (0, 0)),
            pl.BlockSpec((1, B), lambda i: (0, 0)),
        ],
        out_specs=pl.BlockSpec((1, 1), lambda i: (0, 0)),
        out_shape=jax.ShapeDtypeStruct((1, 1), jnp.float32),
        scratch_shapes=[
            pltpu.VMEM((K, B), jnp.float32),
            pltpu.VMEM((K, B), jnp.float32),
            pltpu.VMEM((1, B), jnp.float32),
        ],
        interpret=interpret,
    )(preds_t, t_row)
    return out.reshape((1,))


# speedup vs baseline: 35.8908x; 1.3977x over previous
import functools
import math

import jax
import jax.numpy as jnp
from jax import lax
from jax.experimental import pallas as pl
from jax.experimental.pallas import tpu as pltpu

_EPS = 0.05
_BLK = 512


def _auc_kernel(predsT_ref, trow_ref, out_ref, m1_scr, m2_scr, fac_scr):
    C, B = predsT_ref.shape
    i = pl.program_id(0)
    c2 = math.exp(_EPS)
    c1 = 1.0 + math.exp(2 * _EPS)

    @pl.when(i == 0)
    def _init():
        zt = predsT_ref[...]
        et = jnp.exp(zt - jnp.max(zt, axis=0, keepdims=True))
        pt = et / jnp.sum(et, axis=0, keepdims=True)
        clsr = lax.broadcasted_iota(jnp.int32, (C, B), 0)
        same = trow_ref[...] == clsr
        ohT = same.astype(jnp.float32)
        counts = jnp.sum(ohT, axis=1, keepdims=True)
        denom = counts * (float(B) - counts)
        facc = jnp.where(denom > 0.0, 1.0 / denom, 0.0)
        fac_scr[...] = math.log(2.0) * jnp.sum(ohT * facc, axis=0,
                                               keepdims=True)
        q = jnp.sum(ohT * pt, axis=0, keepdims=True)
        a = c2 * jnp.exp(4.0 - 4.0 * q)
        b = c2 * jnp.exp(4.0 * q - 4.0)
        e4 = jnp.where(same, 0.0, jnp.exp(4.0 * pt))
        e4m = jnp.where(same, 0.0, jnp.exp(-4.0 * pt))
        ones = jnp.ones((1, B), jnp.float32)
        zeros = jnp.zeros((1, B), jnp.float32)
        m1_scr[...] = jnp.concatenate(
            [e4, e4m, c1 * ones, ohT, zeros], axis=0)
        m2_scr[...] = jnp.concatenate(
            [a * ohT, b * ohT, ones, (1.0 - c1) * ohT, zeros], axis=0)
        out_ref[...] = jnp.zeros_like(out_ref)

    ds = pl.ds(i * _BLK, _BLK)
    arg = lax.dot_general(
        m1_scr[...], m2_scr[:, ds],
        dimension_numbers=(((0,), (0,)), ((), ())),
        precision=lax.Precision.DEFAULT)
    colsum = jnp.sum(jnp.log2(arg), axis=0, keepdims=True)
    out_ref[...] += jnp.sum(colsum * fac_scr[:, ds],
                            keepdims=True).reshape(1, 1)


@functools.partial(jax.jit, static_argnames=("interpret",))
def kernel(preds, targets, interpret=False):
    B, C = preds.shape
    K = 3 * C + 2
    preds_t = preds.T
    t_row = targets.astype(jnp.int32).reshape(1, B)
    out = pl.pallas_call(
        _auc_kernel,
        grid=(B // _BLK,),
        in_specs=[
            pl.BlockSpec((C, B), lambda i: (0, 0)),
            pl.BlockSpec((1, B), lambda i: (0, 0)),
        ],
        out_specs=pl.BlockSpec((1, 1), lambda i: (0, 0)),
        out_shape=jax.ShapeDtypeStruct((1, 1), jnp.float32),
        scratch_shapes=[
            pltpu.VMEM((K, B), jnp.float32),
            pltpu.VMEM((K, B), jnp.float32),
            pltpu.VMEM((1, B), jnp.float32),
        ],
        interpret=interpret,
    )(preds_t, t_row)
    return out.reshape((1,))


# speedup vs baseline: 49.0455x; 1.3665x over previous
import functools
import math

import jax
import jax.numpy as jnp
from jax import lax
from jax.experimental import pallas as pl
from jax.experimental.pallas import tpu as pltpu

_EPS = 0.05
_BLK = 2048


def _auc_kernel(predsT_ref, trow_ref, out_ref, m1_scr, m2_scr, fac_scr):
    C, B = predsT_ref.shape
    i = pl.program_id(0)
    c2 = math.exp(_EPS)
    c1 = 1.0 + math.exp(2 * _EPS)

    @pl.when(i == 0)
    def _init():
        zt = predsT_ref[...]
        et = jnp.exp(zt - jnp.max(zt, axis=0, keepdims=True))
        pt = et / jnp.sum(et, axis=0, keepdims=True)
        clsr = lax.broadcasted_iota(jnp.int32, (C, B), 0)
        same = trow_ref[...] == clsr
        ohT = same.astype(jnp.float32)
        counts = jnp.sum(ohT, axis=1, keepdims=True)
        denom = counts * (float(B) - counts)
        facc = jnp.where(denom > 0.0, 1.0 / denom, 0.0)
        fac_scr[...] = jnp.sum(ohT * facc, axis=0, keepdims=True)
        q = jnp.sum(ohT * pt, axis=0, keepdims=True)
        a = c2 * jnp.exp(4.0 - 4.0 * q)
        b = c2 * jnp.exp(4.0 * q - 4.0)
        e4 = jnp.where(same, 0.0, jnp.exp(4.0 * pt))
        e4m = jnp.where(same, 0.0, jnp.exp(-4.0 * pt))
        ones = jnp.ones((1, B), jnp.float32)
        zeros = jnp.zeros((1, B), jnp.float32)
        m1_scr[...] = jnp.concatenate(
            [e4, e4m, c1 * ones, ohT, zeros], axis=0)
        m2_scr[...] = jnp.concatenate(
            [a * ohT, b * ohT, ones, (1.0 - c1) * ohT, zeros], axis=0)
        out_ref[...] = jnp.zeros_like(out_ref)

    ds = pl.ds(i * _BLK, _BLK)
    arg = lax.dot_general(
        m1_scr[...], m2_scr[:, ds],
        dimension_numbers=(((0,), (0,)), ((), ())),
        precision=lax.Precision.DEFAULT)
    R = B // 8
    p0 = arg[0 * R:1 * R] * arg[1 * R:2 * R]
    p1 = arg[2 * R:3 * R] * arg[3 * R:4 * R]
    p2 = arg[4 * R:5 * R] * arg[5 * R:6 * R]
    p3 = arg[6 * R:7 * R] * arg[7 * R:8 * R]
    prod = (p0 * p1) * (p2 * p3)
    colsum = jnp.sum(jnp.log(prod), axis=0, keepdims=True)
    out_ref[...] += jnp.sum(colsum * fac_scr[:, ds],
                            keepdims=True).reshape(1, 1)


@functools.partial(jax.jit, static_argnames=("interpret",))
def kernel(preds, targets, interpret=False):
    B, C = preds.shape
    K = 3 * C + 2
    preds_t = preds.T
    t_row = targets.astype(jnp.int32).reshape(1, B)
    out = pl.pallas_call(
        _auc_kernel,
        grid=(B // _BLK,),
        in_specs=[
            pl.BlockSpec((C, B), lambda i: (0, 0)),
            pl.BlockSpec((1, B), lambda i: (0, 0)),
        ],
        out_specs=pl.BlockSpec((1, 1), lambda i: (0, 0)),
        out_shape=jax.ShapeDtypeStruct((1, 1), jnp.float32),
        scratch_shapes=[
            pltpu.VMEM((K, B), jnp.float32),
            pltpu.VMEM((K, B), jnp.float32),
            pltpu.VMEM((1, B), jnp.float32),
        ],
        interpret=interpret,
    )(preds_t, t_row)
    return out.reshape((1,))
